# Initial kernel scaffold; baseline (speedup 1.0000x reference)
import jax
import jax.numpy as jnp
from jax.experimental import pallas as pl


def kernel(x, pos, token_weight, pos_weight):
    raise NotImplementedError("write your pallas kernel here")


# speedup vs baseline: 1.4467x; 1.4467x over previous
import functools

import jax
import jax.numpy as jnp
from jax import lax
from jax.experimental import pallas as pl
from jax.experimental.pallas import tpu as pltpu
from jax.experimental.pallas import tpu_sc as plsc

D_MODEL = 768
LANES = 16
NUM_CORES = 2
NUM_SUBCORES = 16
NW = NUM_CORES * NUM_SUBCORES


def _make_emb_kernel(n_tot: int):
    per_w = n_tot // NW
    ch = 64
    steps = per_w // ch
    mesh = plsc.VectorSubcoreMesh(core_axis_name="c", subcore_axis_name="s")

    @functools.partial(
        pl.kernel,
        mesh=mesh,
        out_type=jax.ShapeDtypeStruct((n_tot, D_MODEL), jnp.float32),
        scratch_types=[
            pltpu.VMEM((ch,), jnp.int32),
            pltpu.VMEM((ch,), jnp.int32),
            pltpu.VMEM((ch, D_MODEL), jnp.float32),
            pltpu.VMEM((ch, D_MODEL), jnp.float32),
            pltpu.SemaphoreType.DMA,
            pltpu.SemaphoreType.DMA,
        ],
    )
    def emb(x_hbm, p_hbm, tok_hbm, posw_hbm, out_hbm,
            xidx, pidx, arows, brows, sem_a, sem_b):
        wid = lax.axis_index("s") * NUM_CORES + lax.axis_index("c")
        base = wid * per_w

        def step(i, carry):
            off = base + i * ch
            pltpu.sync_copy(x_hbm.at[pl.ds(off, ch)], xidx)
            pltpu.sync_copy(p_hbm.at[pl.ds(off, ch)], pidx)
            ca = pltpu.async_copy(tok_hbm.at[xidx], arows, sem_a)
            cb = pltpu.async_copy(posw_hbm.at[pidx], brows, sem_b)
            ca.wait()
            cb.wait()

            def row(r, rcarry):
                for cc in range(D_MODEL // LANES):
                    sl = pl.ds(cc * LANES, LANES)
                    plsc.addupdate(arows.at[r, sl], brows[r, sl])
                return rcarry

            lax.fori_loop(0, ch, row, 0, unroll=False)
            pltpu.sync_copy(arows, out_hbm.at[pl.ds(off, ch)])
            return carry

        lax.fori_loop(0, steps, step, 0, unroll=False)

    return emb


def kernel(x, pos, token_weight, pos_weight):
    orig_shape = x.shape
    xf = x.reshape(-1).astype(jnp.int32)
    pf = pos.reshape(-1).astype(jnp.int32)
    out = _make_emb_kernel(xf.shape[0])(xf, pf, token_weight, pos_weight)
    return out.reshape(orig_shape + (D_MODEL,))
